# Initial kernel scaffold; baseline (speedup 1.0000x reference)
import jax
import jax.numpy as jnp
from jax.experimental import pallas as pl


def kernel(x):
    raise NotImplementedError("write your pallas kernel here")


# speedup vs baseline: 1.8714x; 1.8714x over previous
import functools

import numpy as np
import jax
import jax.numpy as jnp
from jax.experimental import pallas as pl
from jax.experimental.pallas import tpu as pltpu


def _dft_tables(lf_pad, lf, length):
    f = np.arange(lf_pad, dtype=np.int64)[:, None]
    t = np.arange(length, dtype=np.int64)[None, :]
    ang = (f * t % length).astype(np.float64) * (2.0 * np.pi / length)
    cre = np.cos(ang)
    cim = -np.sin(ang)
    cre[lf:] = 0.0
    cim[lf:] = 0.0
    return jnp.asarray(cre, jnp.float32), jnp.asarray(cim, jnp.float32)


def _mag_body(cre_ref, cim_ref, x_ref, out_ref, *, fb):
    i_f = pl.program_id(0)
    b = pl.program_id(1)
    i_d = pl.program_id(2)
    xb = x_ref[...]
    xre = jnp.dot(cre_ref[...], xb, preferred_element_type=jnp.float32, precision=jax.lax.Precision.HIGHEST)
    xim = jnp.dot(cim_ref[...], xb, preferred_element_type=jnp.float32, precision=jax.lax.Precision.HIGHEST)
    part = jnp.sum(jnp.sqrt(xre * xre + xim * xim), axis=1)[None, None, :]

    @pl.when(i_d == 0)
    def _():
        out_ref[pl.ds(b, 1), pl.ds(i_f, 1), :] = part

    @pl.when(i_d != 0)
    def _():
        out_ref[pl.ds(b, 1), pl.ds(i_f, 1), :] += part


def _topk_body(mag_ref, idx_ref, idxt_ref, *, k):
    m = mag_ref[...]
    B = m.shape[0]
    cols = jax.lax.broadcasted_iota(jnp.int32, m.shape, 1)
    colsk = jax.lax.broadcasted_iota(jnp.int32, (B, k), 1)
    rowsk = jax.lax.broadcasted_iota(jnp.int32, (k, B), 0)
    sentinel = jnp.int32(m.shape[1])

    def body(j, carry):
        m, acc, acct = carry
        mx = jnp.max(m, axis=1, keepdims=True)
        idx = jnp.min(jnp.where(m == mx, cols, sentinel), axis=1)
        acc = jnp.where(colsk == j, idx[:, None], acc)
        acct = jnp.where(rowsk == j, idx[None, :], acct)
        m = jnp.where(cols == idx[:, None], -jnp.inf, m)
        return m, acc, acct

    _, acc, acct = jax.lax.fori_loop(
        0, k, body,
        (m, jnp.zeros((B, k), jnp.int32), jnp.zeros((k, B), jnp.int32)))
    idx_ref[...] = acc
    idxt_ref[...] = acct


def _xsel_body(idxt_ref, x_ref, re_ref, im_ref, *, length):
    b = pl.program_id(0)
    idxt = idxt_ref[...]
    bcols = jax.lax.broadcasted_iota(jnp.int32, idxt.shape, 1)
    fj = jnp.sum(jnp.where(bcols == b, idxt, 0), axis=1, keepdims=True)
    t = jax.lax.broadcasted_iota(jnp.int32, (1, length), 1)
    ang = ((fj * t) & (length - 1)).astype(jnp.float32) * (
        2.0 * np.pi / length)
    xb = x_ref[...]
    re_ref[...] = jnp.dot(jnp.cos(ang), xb, preferred_element_type=jnp.float32, precision=jax.lax.Precision.HIGHEST)
    im_ref[...] = jnp.dot(-jnp.sin(ang), xb, preferred_element_type=jnp.float32, precision=jax.lax.Precision.HIGHEST)


def _inv_body(idx_ref, re_ref, im_ref, y_ref, *, length):
    b = pl.program_id(0)
    idx = idx_ref[...]
    brows = jax.lax.broadcasted_iota(jnp.int32, idx.shape, 0)
    fj = jnp.sum(jnp.where(brows == b, idx, 0), axis=0, keepdims=True)
    t = jax.lax.broadcasted_iota(jnp.int32, (length, 1), 0)
    ang = ((t * fj) & (length - 1)).astype(jnp.float32) * (
        2.0 * np.pi / length)
    w = jnp.where((fj == 0) | (fj == length // 2), 0.5, 1.0) * (2.0 / length)
    a = jnp.cos(ang) * w
    bm = -jnp.sin(ang) * w
    y_ref[...] = (jnp.dot(a, re_ref[...], preferred_element_type=jnp.float32, precision=jax.lax.Precision.HIGHEST)
                  + jnp.dot(bm, im_ref[...], preferred_element_type=jnp.float32, precision=jax.lax.Precision.HIGHEST))


def _pipeline(x, k, interpret=False):
    B, L, D = x.shape
    lf = L // 2 + 1
    lf_pad = -(-lf // 128) * 128
    nf = 4 if (lf_pad % 32 == 0 and lf_pad >= 2048) else 1
    fb = lf_pad // nf
    db1 = 256 if D % 256 == 0 else D
    db2 = 512 if D % 512 == 0 else D

    cre, cim = _dft_tables(lf_pad, lf, L)

    mag = pl.pallas_call(
        functools.partial(_mag_body, fb=fb),
        grid=(nf, B, D // db1),
        in_specs=[
            pl.BlockSpec((fb, L), lambda f, b, d: (f, 0)),
            pl.BlockSpec((fb, L), lambda f, b, d: (f, 0)),
            pl.BlockSpec((None, L, db1), lambda f, b, d: (b, 0, d)),
        ],
        out_specs=pl.BlockSpec((B, nf, fb), lambda f, b, d: (0, 0, 0)),
        out_shape=jax.ShapeDtypeStruct((B, nf, fb), jnp.float32),
        interpret=interpret,
    )(cre, cim, x)
    mag = mag.reshape(B, lf_pad)

    idx, idxt = pl.pallas_call(
        functools.partial(_topk_body, k=k),
        in_specs=[pl.BlockSpec((B, lf_pad), lambda: (0, 0))],
        out_specs=[
            pl.BlockSpec((B, k), lambda: (0, 0)),
            pl.BlockSpec((k, B), lambda: (0, 0)),
        ],
        out_shape=[
            jax.ShapeDtypeStruct((B, k), jnp.int32),
            jax.ShapeDtypeStruct((k, B), jnp.int32),
        ],
        interpret=interpret,
    )(mag)

    re_sel, im_sel = pl.pallas_call(
        functools.partial(_xsel_body, length=L),
        grid=(B, D // db2),
        in_specs=[
            pl.BlockSpec((k, B), lambda b, d: (0, 0)),
            pl.BlockSpec((None, L, db2), lambda b, d: (b, 0, d)),
        ],
        out_specs=[
            pl.BlockSpec((None, k, db2), lambda b, d: (b, 0, d)),
            pl.BlockSpec((None, k, db2), lambda b, d: (b, 0, d)),
        ],
        out_shape=[
            jax.ShapeDtypeStruct((B, k, D), jnp.float32),
            jax.ShapeDtypeStruct((B, k, D), jnp.float32),
        ],
        interpret=interpret,
    )(idxt, x)

    y = pl.pallas_call(
        functools.partial(_inv_body, length=L),
        grid=(B, D // db2),
        in_specs=[
            pl.BlockSpec((B, k), lambda b, d: (0, 0)),
            pl.BlockSpec((None, k, db2), lambda b, d: (b, 0, d)),
            pl.BlockSpec((None, k, db2), lambda b, d: (b, 0, d)),
        ],
        out_specs=pl.BlockSpec((None, L, db2), lambda b, d: (b, 0, d)),
        out_shape=jax.ShapeDtypeStruct((B, L, D), jnp.float32),
        interpret=interpret,
    )(idx, re_sel, im_sel)

    return y


def kernel(x):
    return _pipeline(x, 64)


# speedup vs baseline: 3.0858x; 1.6490x over previous
import functools

import numpy as np
import jax
import jax.numpy as jnp
from jax.experimental import pallas as pl
from jax.experimental.pallas import tpu as pltpu


def _dft_tables(lf_pad, lf, length):
    f = np.arange(lf_pad, dtype=np.int64)[:, None]
    t = np.arange(length, dtype=np.int64)[None, :]
    ang = (f * t % length).astype(np.float64) * (2.0 * np.pi / length)
    cre = np.cos(ang)
    cim = -np.sin(ang)
    cre[lf:] = 0.0
    cim[lf:] = 0.0

    def split(a):
        a32 = a.astype(np.float32)
        hi = a32.astype(jnp.bfloat16)
        lo = (a32 - hi.astype(np.float32)).astype(jnp.bfloat16)
        return jnp.asarray(hi), jnp.asarray(lo)

    return split(cre) + split(cim)


def _mag_body(cre_h_ref, cre_l_ref, cim_h_ref, cim_l_ref, x_ref, out_ref, *, fb):
    i_f = pl.program_id(0)
    b = pl.program_id(1)
    i_d = pl.program_id(2)
    xb = x_ref[...]
    xh = xb.astype(jnp.bfloat16)
    xl = (xb - xh.astype(jnp.float32)).astype(jnp.bfloat16)

    def mm3(h_ref, l_ref):
        h = h_ref[...]
        out = jnp.dot(h, xh, preferred_element_type=jnp.float32)
        out += jnp.dot(h, xl, preferred_element_type=jnp.float32)
        out += jnp.dot(l_ref[...], xh, preferred_element_type=jnp.float32)
        return out

    xre = mm3(cre_h_ref, cre_l_ref)
    xim = mm3(cim_h_ref, cim_l_ref)
    part = jnp.sum(jnp.sqrt(xre * xre + xim * xim), axis=1)[None, None, :]

    @pl.when(i_d == 0)
    def _():
        out_ref[pl.ds(b, 1), pl.ds(i_f, 1), :] = part

    @pl.when(i_d != 0)
    def _():
        out_ref[pl.ds(b, 1), pl.ds(i_f, 1), :] += part


def _topk_body(mag_ref, idx_ref, idxt_ref, *, k):
    m = mag_ref[...]
    B = m.shape[0]
    cols = jax.lax.broadcasted_iota(jnp.int32, m.shape, 1)
    colsk = jax.lax.broadcasted_iota(jnp.int32, (B, k), 1)
    rowsk = jax.lax.broadcasted_iota(jnp.int32, (k, B), 0)
    sentinel = jnp.int32(m.shape[1])

    def body(j, carry):
        m, acc, acct = carry
        mx = jnp.max(m, axis=1, keepdims=True)
        idx = jnp.min(jnp.where(m == mx, cols, sentinel), axis=1)
        acc = jnp.where(colsk == j, idx[:, None], acc)
        acct = jnp.where(rowsk == j, idx[None, :], acct)
        m = jnp.where(cols == idx[:, None], -jnp.inf, m)
        return m, acc, acct

    _, acc, acct = jax.lax.fori_loop(
        0, k, body,
        (m, jnp.zeros((B, k), jnp.int32), jnp.zeros((k, B), jnp.int32)))
    idx_ref[...] = acc
    idxt_ref[...] = acct


def _xsel_body(idxt_ref, x_ref, re_ref, im_ref, *, length):
    b = pl.program_id(0)
    idxt = idxt_ref[...]
    bcols = jax.lax.broadcasted_iota(jnp.int32, idxt.shape, 1)
    fj = jnp.sum(jnp.where(bcols == b, idxt, 0), axis=1, keepdims=True)
    t = jax.lax.broadcasted_iota(jnp.int32, (1, length), 1)
    ang = ((fj * t) & (length - 1)).astype(jnp.float32) * (
        2.0 * np.pi / length)
    xb = x_ref[...]
    re_ref[...] = jnp.dot(jnp.cos(ang), xb, preferred_element_type=jnp.float32, precision=jax.lax.Precision.HIGHEST)
    im_ref[...] = jnp.dot(-jnp.sin(ang), xb, preferred_element_type=jnp.float32, precision=jax.lax.Precision.HIGHEST)


def _inv_body(idx_ref, re_ref, im_ref, y_ref, *, length):
    b = pl.program_id(0)
    idx = idx_ref[...]
    brows = jax.lax.broadcasted_iota(jnp.int32, idx.shape, 0)
    fj = jnp.sum(jnp.where(brows == b, idx, 0), axis=0, keepdims=True)
    t = jax.lax.broadcasted_iota(jnp.int32, (length, 1), 0)
    ang = ((t * fj) & (length - 1)).astype(jnp.float32) * (
        2.0 * np.pi / length)
    w = jnp.where((fj == 0) | (fj == length // 2), 0.5, 1.0) * (2.0 / length)
    a = jnp.cos(ang) * w
    bm = -jnp.sin(ang) * w
    y_ref[...] = (jnp.dot(a, re_ref[...], preferred_element_type=jnp.float32, precision=jax.lax.Precision.HIGHEST)
                  + jnp.dot(bm, im_ref[...], preferred_element_type=jnp.float32, precision=jax.lax.Precision.HIGHEST))


def _pipeline(x, k, interpret=False):
    B, L, D = x.shape
    lf = L // 2 + 1
    lf_pad = -(-lf // 128) * 128
    nf = 4 if (lf_pad % 32 == 0 and lf_pad >= 2048) else 1
    fb = lf_pad // nf
    db1 = 256 if D % 256 == 0 else D
    db2 = 512 if D % 512 == 0 else D

    cre_h, cre_l, cim_h, cim_l = _dft_tables(lf_pad, lf, L)

    basis_spec = pl.BlockSpec((fb, L), lambda f, b, d: (f, 0))
    mag = pl.pallas_call(
        functools.partial(_mag_body, fb=fb),
        grid=(nf, B, D // db1),
        in_specs=[
            basis_spec, basis_spec, basis_spec, basis_spec,
            pl.BlockSpec((None, L, db1), lambda f, b, d: (b, 0, d)),
        ],
        out_specs=pl.BlockSpec((B, nf, fb), lambda f, b, d: (0, 0, 0)),
        out_shape=jax.ShapeDtypeStruct((B, nf, fb), jnp.float32),
        interpret=interpret,
    )(cre_h, cre_l, cim_h, cim_l, x)
    mag = mag.reshape(B, lf_pad)

    idx, idxt = pl.pallas_call(
        functools.partial(_topk_body, k=k),
        in_specs=[pl.BlockSpec((B, lf_pad), lambda: (0, 0))],
        out_specs=[
            pl.BlockSpec((B, k), lambda: (0, 0)),
            pl.BlockSpec((k, B), lambda: (0, 0)),
        ],
        out_shape=[
            jax.ShapeDtypeStruct((B, k), jnp.int32),
            jax.ShapeDtypeStruct((k, B), jnp.int32),
        ],
        interpret=interpret,
    )(mag)

    re_sel, im_sel = pl.pallas_call(
        functools.partial(_xsel_body, length=L),
        grid=(B, D // db2),
        in_specs=[
            pl.BlockSpec((k, B), lambda b, d: (0, 0)),
            pl.BlockSpec((None, L, db2), lambda b, d: (b, 0, d)),
        ],
        out_specs=[
            pl.BlockSpec((None, k, db2), lambda b, d: (b, 0, d)),
            pl.BlockSpec((None, k, db2), lambda b, d: (b, 0, d)),
        ],
        out_shape=[
            jax.ShapeDtypeStruct((B, k, D), jnp.float32),
            jax.ShapeDtypeStruct((B, k, D), jnp.float32),
        ],
        interpret=interpret,
    )(idxt, x)

    y = pl.pallas_call(
        functools.partial(_inv_body, length=L),
        grid=(B, D // db2),
        in_specs=[
            pl.BlockSpec((B, k), lambda b, d: (0, 0)),
            pl.BlockSpec((None, k, db2), lambda b, d: (b, 0, d)),
            pl.BlockSpec((None, k, db2), lambda b, d: (b, 0, d)),
        ],
        out_specs=pl.BlockSpec((None, L, db2), lambda b, d: (b, 0, d)),
        out_shape=jax.ShapeDtypeStruct((B, L, D), jnp.float32),
        interpret=interpret,
    )(idx, re_sel, im_sel)

    return y


def kernel(x):
    return _pipeline(x, 64)


# speedup vs baseline: 3.3476x; 1.0848x over previous
import functools

import numpy as np
import jax
import jax.numpy as jnp
from jax.experimental import pallas as pl
from jax.experimental.pallas import tpu as pltpu


def _dft_tables(lf_pad, lf, length):
    f = np.arange(lf_pad, dtype=np.int64)[:, None]
    t = np.arange(lf_pad, dtype=np.int64)[None, :]
    ang = (f * t % length).astype(np.float64) * (2.0 * np.pi / length)
    cre = np.cos(ang)
    cim = -np.sin(ang)
    cre[lf:] = 0.0
    cim[lf:] = 0.0
    cre[:, lf:] = 0.0
    cim[:, lf:] = 0.0
    cre[:, 0] *= 0.5
    if length // 2 < lf_pad:
        cre[:, length // 2] *= 0.5
    cim[:, 0] = 0.0
    if length // 2 < lf_pad:
        cim[:, length // 2] = 0.0

    def split(a):
        a32 = a.astype(np.float32)
        hi = a32.astype(jnp.bfloat16)
        lo = (a32 - hi.astype(np.float32)).astype(jnp.bfloat16)
        return jnp.asarray(hi), jnp.asarray(lo)

    return split(cre) + split(cim)


def _mag_body(cre_h_ref, cre_l_ref, cim_h_ref, cim_l_ref, xa_ref, xr_ref,
              out_ref, *, fb):
    i_f = pl.program_id(0)
    b = pl.program_id(1)
    i_d = pl.program_id(2)
    xa = xa_ref[...]
    xr = xr_ref[...]
    xe = xa + xr
    xo = xa - xr

    def mm3(h_ref, l_ref, v):
        vh = v.astype(jnp.bfloat16)
        vl = (v - vh.astype(jnp.float32)).astype(jnp.bfloat16)
        h = h_ref[...]
        out = jnp.dot(h, vh, preferred_element_type=jnp.float32)
        out += jnp.dot(h, vl, preferred_element_type=jnp.float32)
        out += jnp.dot(l_ref[...], vh, preferred_element_type=jnp.float32)
        return out

    xre = mm3(cre_h_ref, cre_l_ref, xe)
    xim = mm3(cim_h_ref, cim_l_ref, xo)
    part = jnp.sum(jnp.sqrt(xre * xre + xim * xim), axis=1)[None, None, :]

    @pl.when(i_d == 0)
    def _():
        out_ref[pl.ds(b, 1), pl.ds(i_f, 1), :] = part

    @pl.when(i_d != 0)
    def _():
        out_ref[pl.ds(b, 1), pl.ds(i_f, 1), :] += part


def _topk_body(mag_ref, idx_ref, idxt_ref, *, k):
    m = mag_ref[...]
    B = m.shape[0]
    cols = jax.lax.broadcasted_iota(jnp.int32, m.shape, 1)
    colsk = jax.lax.broadcasted_iota(jnp.int32, (B, k), 1)
    rowsk = jax.lax.broadcasted_iota(jnp.int32, (k, B), 0)
    sentinel = jnp.int32(m.shape[1])

    def body(j, carry):
        m, acc, acct = carry
        mx = jnp.max(m, axis=1, keepdims=True)
        idx = jnp.min(jnp.where(m == mx, cols, sentinel), axis=1)
        acc = jnp.where(colsk == j, idx[:, None], acc)
        acct = jnp.where(rowsk == j, idx[None, :], acct)
        m = jnp.where(cols == idx[:, None], -jnp.inf, m)
        return m, acc, acct

    _, acc, acct = jax.lax.fori_loop(
        0, k, body,
        (m, jnp.zeros((B, k), jnp.int32), jnp.zeros((k, B), jnp.int32)))
    idx_ref[...] = acc
    idxt_ref[...] = acct


def _xsel_body(idxt_ref, x_ref, re_ref, im_ref, *, length):
    b = pl.program_id(0)
    idxt = idxt_ref[...]
    bcols = jax.lax.broadcasted_iota(jnp.int32, idxt.shape, 1)
    fj = jnp.sum(jnp.where(bcols == b, idxt, 0), axis=1, keepdims=True)
    t = jax.lax.broadcasted_iota(jnp.int32, (1, length), 1)
    ang = ((fj * t) & (length - 1)).astype(jnp.float32) * (
        2.0 * np.pi / length)
    xb = x_ref[...]
    re_ref[...] = jnp.dot(jnp.cos(ang), xb, preferred_element_type=jnp.float32, precision=jax.lax.Precision.HIGHEST)
    im_ref[...] = jnp.dot(-jnp.sin(ang), xb, preferred_element_type=jnp.float32, precision=jax.lax.Precision.HIGHEST)


def _inv_body(idx_ref, re_ref, im_ref, y_ref, *, length):
    b = pl.program_id(0)
    idx = idx_ref[...]
    brows = jax.lax.broadcasted_iota(jnp.int32, idx.shape, 0)
    fj = jnp.sum(jnp.where(brows == b, idx, 0), axis=0, keepdims=True)
    t = jax.lax.broadcasted_iota(jnp.int32, (length, 1), 0)
    ang = ((t * fj) & (length - 1)).astype(jnp.float32) * (
        2.0 * np.pi / length)
    w = jnp.where((fj == 0) | (fj == length // 2), 0.5, 1.0) * (2.0 / length)
    a = jnp.cos(ang) * w
    bm = -jnp.sin(ang) * w
    y_ref[...] = (jnp.dot(a, re_ref[...], preferred_element_type=jnp.float32, precision=jax.lax.Precision.HIGHEST)
                  + jnp.dot(bm, im_ref[...], preferred_element_type=jnp.float32, precision=jax.lax.Precision.HIGHEST))


def _pipeline(x, k, interpret=False):
    B, L, D = x.shape
    lf = L // 2 + 1
    lf_pad = -(-lf // 128) * 128
    nf = 2 if (lf_pad % 16 == 0 and lf_pad >= 2048) else 1
    fb = lf_pad // nf
    db1 = 256 if D % 256 == 0 else D
    db2 = 512 if D % 512 == 0 else D

    cre_h, cre_l, cim_h, cim_l = _dft_tables(lf_pad, lf, L)

    t_rev = (-jnp.arange(lf)) % L
    pad = [(0, 0), (0, lf_pad - lf), (0, 0)]
    xa = jnp.pad(x[:, :lf, :], pad)
    xr = jnp.pad(jnp.take(x, t_rev, axis=1), pad)

    basis_spec = pl.BlockSpec((fb, lf_pad), lambda f, b, d: (f, 0))
    xfold_spec = pl.BlockSpec((None, lf_pad, db1), lambda f, b, d: (b, 0, d))
    mag = pl.pallas_call(
        functools.partial(_mag_body, fb=fb),
        grid=(nf, B, D // db1),
        in_specs=[
            basis_spec, basis_spec, basis_spec, basis_spec,
            xfold_spec, xfold_spec,
        ],
        out_specs=pl.BlockSpec((B, nf, fb), lambda f, b, d: (0, 0, 0)),
        out_shape=jax.ShapeDtypeStruct((B, nf, fb), jnp.float32),
        interpret=interpret,
    )(cre_h, cre_l, cim_h, cim_l, xa, xr)
    mag = mag.reshape(B, lf_pad)

    idx, idxt = pl.pallas_call(
        functools.partial(_topk_body, k=k),
        in_specs=[pl.BlockSpec((B, lf_pad), lambda: (0, 0))],
        out_specs=[
            pl.BlockSpec((B, k), lambda: (0, 0)),
            pl.BlockSpec((k, B), lambda: (0, 0)),
        ],
        out_shape=[
            jax.ShapeDtypeStruct((B, k), jnp.int32),
            jax.ShapeDtypeStruct((k, B), jnp.int32),
        ],
        interpret=interpret,
    )(mag)

    re_sel, im_sel = pl.pallas_call(
        functools.partial(_xsel_body, length=L),
        grid=(B, D // db2),
        in_specs=[
            pl.BlockSpec((k, B), lambda b, d: (0, 0)),
            pl.BlockSpec((None, L, db2), lambda b, d: (b, 0, d)),
        ],
        out_specs=[
            pl.BlockSpec((None, k, db2), lambda b, d: (b, 0, d)),
            pl.BlockSpec((None, k, db2), lambda b, d: (b, 0, d)),
        ],
        out_shape=[
            jax.ShapeDtypeStruct((B, k, D), jnp.float32),
            jax.ShapeDtypeStruct((B, k, D), jnp.float32),
        ],
        interpret=interpret,
    )(idxt, x)

    y = pl.pallas_call(
        functools.partial(_inv_body, length=L),
        grid=(B, D // db2),
        in_specs=[
            pl.BlockSpec((B, k), lambda b, d: (0, 0)),
            pl.BlockSpec((None, k, db2), lambda b, d: (b, 0, d)),
            pl.BlockSpec((None, k, db2), lambda b, d: (b, 0, d)),
        ],
        out_specs=pl.BlockSpec((None, L, db2), lambda b, d: (b, 0, d)),
        out_shape=jax.ShapeDtypeStruct((B, L, D), jnp.float32),
        interpret=interpret,
    )(idx, re_sel, im_sel)

    return y


def kernel(x):
    return _pipeline(x, 64)


# speedup vs baseline: 4.0640x; 1.2140x over previous
import functools

import numpy as np
import jax
import jax.numpy as jnp
from jax.experimental import pallas as pl
from jax.experimental.pallas import tpu as pltpu


def _dft_tables(lf_pad, lf, length):
    f = np.arange(lf_pad, dtype=np.int64)[:, None]
    t = np.arange(lf_pad, dtype=np.int64)[None, :]
    ang = (f * t % length).astype(np.float64) * (2.0 * np.pi / length)
    cre = np.cos(ang)
    cim = -np.sin(ang)
    cre[lf:] = 0.0
    cim[lf:] = 0.0
    cre[:, lf:] = 0.0
    cim[:, lf:] = 0.0
    cre[:, 0] *= 0.5
    if length // 2 < lf_pad:
        cre[:, length // 2] *= 0.5
    cim[:, 0] = 0.0
    if length // 2 < lf_pad:
        cim[:, length // 2] = 0.0

    def split(a):
        a32 = a.astype(np.float32)
        hi = a32.astype(jnp.bfloat16)
        lo = (a32 - hi.astype(np.float32)).astype(jnp.bfloat16)
        return jnp.asarray(hi), jnp.asarray(lo)

    return split(cre) + split(cim)


def _mag_body(cre_h_ref, cre_l_ref, cim_h_ref, cim_l_ref, xa_ref, xr_ref,
              out_ref, *, fb):
    i_f = pl.program_id(0)
    b = pl.program_id(1)
    i_d = pl.program_id(2)
    xa = xa_ref[...]
    xr = xr_ref[...]
    xe = xa + xr
    xo = xa - xr

    def mm3(h_ref, l_ref, v):
        vh = v.astype(jnp.bfloat16)
        vl = (v - vh.astype(jnp.float32)).astype(jnp.bfloat16)
        h = h_ref[...]
        out = jnp.dot(h, vh, preferred_element_type=jnp.float32)
        out += jnp.dot(h, vl, preferred_element_type=jnp.float32)
        out += jnp.dot(l_ref[...], vh, preferred_element_type=jnp.float32)
        return out

    xre = mm3(cre_h_ref, cre_l_ref, xe)
    xim = mm3(cim_h_ref, cim_l_ref, xo)
    part = jnp.sum(jnp.sqrt(xre * xre + xim * xim), axis=1)[None, None, :]

    @pl.when(i_d == 0)
    def _():
        out_ref[pl.ds(b, 1), pl.ds(i_f, 1), :] = part

    @pl.when(i_d != 0)
    def _():
        out_ref[pl.ds(b, 1), pl.ds(i_f, 1), :] += part


def _topk_body(mag_ref, idx_ref, idxt_ref, *, k):
    m = mag_ref[...]
    B = m.shape[0]
    cols = jax.lax.broadcasted_iota(jnp.int32, m.shape, 1)
    colsk = jax.lax.broadcasted_iota(jnp.int32, (B, k), 1)
    rowsk = jax.lax.broadcasted_iota(jnp.int32, (k, B), 0)
    sentinel = jnp.int32(m.shape[1])

    def body(j, carry):
        m, acc, acct = carry
        mx = jnp.max(m, axis=1, keepdims=True)
        idx = jnp.min(jnp.where(m == mx, cols, sentinel), axis=1)
        acc = jnp.where(colsk == j, idx[:, None], acc)
        acct = jnp.where(rowsk == j, idx[None, :], acct)
        m = jnp.where(cols == idx[:, None], -jnp.inf, m)
        return m, acc, acct

    _, acc, acct = jax.lax.fori_loop(
        0, k, body,
        (m, jnp.zeros((B, k), jnp.int32), jnp.zeros((k, B), jnp.int32)))
    idx_ref[...] = acc
    idxt_ref[...] = acct


def _split_bf16(v):
    vh = v.astype(jnp.bfloat16)
    vl = (v - vh.astype(jnp.float32)).astype(jnp.bfloat16)
    return vh, vl


def _dot3(c, x):
    ch, cl = _split_bf16(c)
    xh, xl = _split_bf16(x)
    out = jnp.dot(ch, xh, preferred_element_type=jnp.float32)
    out += jnp.dot(ch, xl, preferred_element_type=jnp.float32)
    out += jnp.dot(cl, xh, preferred_element_type=jnp.float32)
    return out


def _xsel_body(idxt_ref, x_ref, sel_ref, *, length, k):
    b = pl.program_id(0)
    idxt = idxt_ref[...]
    bcols = jax.lax.broadcasted_iota(jnp.int32, idxt.shape, 1)
    fj = jnp.sum(jnp.where(bcols == b, idxt, 0), axis=1, keepdims=True)
    fj2 = jnp.concatenate([fj, fj], axis=0)
    t = jax.lax.broadcasted_iota(jnp.int32, (1, length), 1)
    ang = ((fj2 * t) & (length - 1)).astype(jnp.float32) * (
        2.0 * np.pi / length)
    is_im = jax.lax.broadcasted_iota(jnp.int32, (2 * k, 1), 0) >= k
    c = jnp.where(is_im, -jnp.sin(ang), jnp.cos(ang))
    sel_ref[...] = _dot3(c, x_ref[...])


def _inv_body(idx_ref, sel_ref, y_ref, *, length, k):
    b = pl.program_id(0)
    idx = idx_ref[...]
    brows = jax.lax.broadcasted_iota(jnp.int32, idx.shape, 0)
    fj = jnp.sum(jnp.where(brows == b, idx, 0), axis=0, keepdims=True)
    fj2 = jnp.concatenate([fj, fj], axis=1)
    t = jax.lax.broadcasted_iota(jnp.int32, (length, 1), 0)
    ang = ((t * fj2) & (length - 1)).astype(jnp.float32) * (
        2.0 * np.pi / length)
    w = jnp.where((fj2 == 0) | (fj2 == length // 2), 0.5, 1.0) * (2.0 / length)
    is_im = jax.lax.broadcasted_iota(jnp.int32, (1, 2 * k), 1) >= k
    c = jnp.where(is_im, -jnp.sin(ang), jnp.cos(ang)) * w
    y_ref[...] = _dot3(c, sel_ref[...])


def _pipeline(x, k, interpret=False, upto=4):
    B, L, D = x.shape
    lf = L // 2 + 1
    lf_pad = -(-lf // 128) * 128
    nf = 2 if (lf_pad % 16 == 0 and lf_pad >= 2048) else 1
    fb = lf_pad // nf
    db1 = 256 if D % 256 == 0 else D
    db2 = 512 if D % 512 == 0 else D

    cre_h, cre_l, cim_h, cim_l = _dft_tables(lf_pad, lf, L)

    t_rev = (-jnp.arange(lf)) % L
    pad = [(0, 0), (0, lf_pad - lf), (0, 0)]
    xa = jnp.pad(x[:, :lf, :], pad)
    xr = jnp.pad(jnp.take(x, t_rev, axis=1), pad)

    basis_spec = pl.BlockSpec((fb, lf_pad), lambda f, b, d: (f, 0))
    xfold_spec = pl.BlockSpec((None, lf_pad, db1), lambda f, b, d: (b, 0, d))
    mag = pl.pallas_call(
        functools.partial(_mag_body, fb=fb),
        grid=(nf, B, D // db1),
        in_specs=[
            basis_spec, basis_spec, basis_spec, basis_spec,
            xfold_spec, xfold_spec,
        ],
        out_specs=pl.BlockSpec((B, nf, fb), lambda f, b, d: (0, 0, 0)),
        out_shape=jax.ShapeDtypeStruct((B, nf, fb), jnp.float32),
        interpret=interpret,
    )(cre_h, cre_l, cim_h, cim_l, xa, xr)
    mag = mag.reshape(B, lf_pad)
    if upto == 1:
        return mag

    idx, idxt = pl.pallas_call(
        functools.partial(_topk_body, k=k),
        in_specs=[pl.BlockSpec((B, lf_pad), lambda: (0, 0))],
        out_specs=[
            pl.BlockSpec((B, k), lambda: (0, 0)),
            pl.BlockSpec((k, B), lambda: (0, 0)),
        ],
        out_shape=[
            jax.ShapeDtypeStruct((B, k), jnp.int32),
            jax.ShapeDtypeStruct((k, B), jnp.int32),
        ],
        interpret=interpret,
    )(mag)
    if upto == 2:
        return idx

    sel = pl.pallas_call(
        functools.partial(_xsel_body, length=L, k=k),
        grid=(B, D // db2),
        in_specs=[
            pl.BlockSpec((k, B), lambda b, d: (0, 0)),
            pl.BlockSpec((None, L, db2), lambda b, d: (b, 0, d)),
        ],
        out_specs=pl.BlockSpec((None, 2 * k, db2), lambda b, d: (b, 0, d)),
        out_shape=jax.ShapeDtypeStruct((B, 2 * k, D), jnp.float32),
        interpret=interpret,
    )(idxt, x)
    if upto == 3:
        return sel

    y = pl.pallas_call(
        functools.partial(_inv_body, length=L, k=k),
        grid=(B, D // db2),
        in_specs=[
            pl.BlockSpec((B, k), lambda b, d: (0, 0)),
            pl.BlockSpec((None, 2 * k, db2), lambda b, d: (b, 0, d)),
        ],
        out_specs=pl.BlockSpec((None, L, db2), lambda b, d: (b, 0, d)),
        out_shape=jax.ShapeDtypeStruct((B, L, D), jnp.float32),
        interpret=interpret,
    )(idx, sel)

    return y


def kernel(x):
    return _pipeline(x, 64)


# speedup vs baseline: 4.4987x; 1.1070x over previous
import functools

import numpy as np
import jax
import jax.numpy as jnp
from jax.experimental import pallas as pl
from jax.experimental.pallas import tpu as pltpu


def _dft_tables(lf_pad, lf, length):
    f = np.arange(lf_pad, dtype=np.int64)[:, None]
    t = np.arange(lf_pad, dtype=np.int64)[None, :]
    ang = (f * t % length).astype(np.float64) * (2.0 * np.pi / length)
    cre = np.cos(ang)
    cim = -np.sin(ang)
    cre[lf:] = 0.0
    cim[lf:] = 0.0
    cre[:, lf:] = 0.0
    cim[:, lf:] = 0.0
    cre[:, 0] *= 0.5
    if length // 2 < lf_pad:
        cre[:, length // 2] *= 0.5
    cim[:, 0] = 0.0
    if length // 2 < lf_pad:
        cim[:, length // 2] = 0.0

    def split(a):
        a32 = a.astype(np.float32)
        hi = a32.astype(jnp.bfloat16)
        lo = (a32 - hi.astype(np.float32)).astype(jnp.bfloat16)
        return jnp.asarray(hi), jnp.asarray(lo)

    return split(cre) + split(cim)


def _mag_body(cre_h_ref, cre_l_ref, cim_h_ref, cim_l_ref, xa_ref, xr_ref,
              out_ref, *, fb):
    i_f = pl.program_id(0)
    b = pl.program_id(1)
    i_d = pl.program_id(2)
    xa = xa_ref[...]
    xr = xr_ref[...]
    xe = xa + xr
    xo = xa - xr

    def mm3(h_ref, l_ref, v):
        vh = v.astype(jnp.bfloat16)
        vl = (v - vh.astype(jnp.float32)).astype(jnp.bfloat16)
        h = h_ref[...]
        out = jnp.dot(h, vh, preferred_element_type=jnp.float32)
        out += jnp.dot(h, vl, preferred_element_type=jnp.float32)
        out += jnp.dot(l_ref[...], vh, preferred_element_type=jnp.float32)
        return out

    xre = mm3(cre_h_ref, cre_l_ref, xe)
    xim = mm3(cim_h_ref, cim_l_ref, xo)
    part = jnp.sum(jnp.sqrt(xre * xre + xim * xim), axis=1)[None, None, :]

    @pl.when(i_d == 0)
    def _():
        out_ref[pl.ds(b, 1), pl.ds(i_f, 1), :] = part

    @pl.when(i_d != 0)
    def _():
        out_ref[pl.ds(b, 1), pl.ds(i_f, 1), :] += part


def _topk_body(mag_ref, idx_ref, idxt_ref, *, k):
    m = mag_ref[...]
    B = m.shape[0]
    cols = jax.lax.broadcasted_iota(jnp.int32, m.shape, 1)
    colsk = jax.lax.broadcasted_iota(jnp.int32, (B, k), 1)
    rowsk = jax.lax.broadcasted_iota(jnp.int32, (k, B), 0)
    sentinel = jnp.int32(m.shape[1])

    def body(j, carry):
        m, acc, acct = carry
        mx = jnp.max(m, axis=1, keepdims=True)
        idx = jnp.min(jnp.where(m == mx, cols, sentinel), axis=1)
        acc = jnp.where(colsk == j, idx[:, None], acc)
        acct = jnp.where(rowsk == j, idx[None, :], acct)
        m = jnp.where(cols == idx[:, None], -jnp.inf, m)
        return m, acc, acct

    _, acc, acct = jax.lax.fori_loop(
        0, k, body,
        (m, jnp.zeros((B, k), jnp.int32), jnp.zeros((k, B), jnp.int32)))
    idx_ref[...] = acc
    idxt_ref[...] = acct


def _split_bf16(v):
    vh = v.astype(jnp.bfloat16)
    vl = (v - vh.astype(jnp.float32)).astype(jnp.bfloat16)
    return vh, vl


def _dot3(c, x):
    ch, cl = _split_bf16(c)
    xh, xl = _split_bf16(x)
    out = jnp.dot(ch, xh, preferred_element_type=jnp.float32)
    out += jnp.dot(ch, xl, preferred_element_type=jnp.float32)
    out += jnp.dot(cl, xh, preferred_element_type=jnp.float32)
    return out


def _xsel_body(idxt_ref, x_ref, sel_ref, bh_ref, bl_ref, *, length, k):
    b = pl.program_id(0)

    @pl.when(pl.program_id(1) == 0)
    def _():
        idxt = idxt_ref[...]
        bcols = jax.lax.broadcasted_iota(jnp.int32, idxt.shape, 1)
        fj = jnp.sum(jnp.where(bcols == b, idxt, 0), axis=1, keepdims=True)
        fj2 = jnp.concatenate([fj, fj], axis=0)
        t = jax.lax.broadcasted_iota(jnp.int32, (1, length), 1)
        ang = ((fj2 * t) & (length - 1)).astype(jnp.float32) * (
            2.0 * np.pi / length)
        is_im = jax.lax.broadcasted_iota(jnp.int32, (2 * k, 1), 0) >= k
        c = jnp.where(is_im, -jnp.sin(ang), jnp.cos(ang))
        ch, cl = _split_bf16(c)
        bh_ref[...] = ch
        bl_ref[...] = cl

    xh, xl = _split_bf16(x_ref[...])
    ch = bh_ref[...]
    out = jnp.dot(ch, xh, preferred_element_type=jnp.float32)
    out += jnp.dot(ch, xl, preferred_element_type=jnp.float32)
    out += jnp.dot(bl_ref[...], xh, preferred_element_type=jnp.float32)
    sel_ref[...] = out


def _inv_body(idx_ref, sel_ref, y_ref, bh_ref, bl_ref, *, length, k):
    b = pl.program_id(0)

    @pl.when(pl.program_id(1) == 0)
    def _():
        idx = idx_ref[...]
        brows = jax.lax.broadcasted_iota(jnp.int32, idx.shape, 0)
        fj = jnp.sum(jnp.where(brows == b, idx, 0), axis=0, keepdims=True)
        fj2 = jnp.concatenate([fj, fj], axis=1)
        t = jax.lax.broadcasted_iota(jnp.int32, (length, 1), 0)
        ang = ((t * fj2) & (length - 1)).astype(jnp.float32) * (
            2.0 * np.pi / length)
        w = jnp.where((fj2 == 0) | (fj2 == length // 2), 0.5, 1.0) * (
            2.0 / length)
        is_im = jax.lax.broadcasted_iota(jnp.int32, (1, 2 * k), 1) >= k
        c = jnp.where(is_im, -jnp.sin(ang), jnp.cos(ang)) * w
        ch, cl = _split_bf16(c)
        bh_ref[...] = ch
        bl_ref[...] = cl

    xh, xl = _split_bf16(sel_ref[...])
    ch = bh_ref[...]
    out = jnp.dot(ch, xh, preferred_element_type=jnp.float32)
    out += jnp.dot(ch, xl, preferred_element_type=jnp.float32)
    out += jnp.dot(bl_ref[...], xh, preferred_element_type=jnp.float32)
    y_ref[...] = out


def _pipeline(x, k, interpret=False, upto=4):
    B, L, D = x.shape
    lf = L // 2 + 1
    lf_pad = -(-lf // 128) * 128
    nf = 2 if (lf_pad % 16 == 0 and lf_pad >= 2048) else 1
    fb = lf_pad // nf
    db1 = 256 if D % 256 == 0 else D
    db2 = 512 if D % 512 == 0 else D

    cre_h, cre_l, cim_h, cim_l = _dft_tables(lf_pad, lf, L)

    t_rev = (-jnp.arange(lf)) % L
    pad = [(0, 0), (0, lf_pad - lf), (0, 0)]
    xa = jnp.pad(x[:, :lf, :], pad)
    xr = jnp.pad(jnp.take(x, t_rev, axis=1), pad)

    basis_spec = pl.BlockSpec((fb, lf_pad), lambda f, b, d: (f, 0))
    xfold_spec = pl.BlockSpec((None, lf_pad, db1), lambda f, b, d: (b, 0, d))
    mag = pl.pallas_call(
        functools.partial(_mag_body, fb=fb),
        grid=(nf, B, D // db1),
        in_specs=[
            basis_spec, basis_spec, basis_spec, basis_spec,
            xfold_spec, xfold_spec,
        ],
        out_specs=pl.BlockSpec((B, nf, fb), lambda f, b, d: (0, 0, 0)),
        out_shape=jax.ShapeDtypeStruct((B, nf, fb), jnp.float32),
        interpret=interpret,
    )(cre_h, cre_l, cim_h, cim_l, xa, xr)
    mag = mag.reshape(B, lf_pad)
    if upto == 1:
        return mag

    idx, idxt = pl.pallas_call(
        functools.partial(_topk_body, k=k),
        in_specs=[pl.BlockSpec((B, lf_pad), lambda: (0, 0))],
        out_specs=[
            pl.BlockSpec((B, k), lambda: (0, 0)),
            pl.BlockSpec((k, B), lambda: (0, 0)),
        ],
        out_shape=[
            jax.ShapeDtypeStruct((B, k), jnp.int32),
            jax.ShapeDtypeStruct((k, B), jnp.int32),
        ],
        interpret=interpret,
    )(mag)
    if upto == 2:
        return idx

    sel = pl.pallas_call(
        functools.partial(_xsel_body, length=L, k=k),
        grid=(B, D // db2),
        in_specs=[
            pl.BlockSpec((k, B), lambda b, d: (0, 0)),
            pl.BlockSpec((None, L, db2), lambda b, d: (b, 0, d)),
        ],
        out_specs=pl.BlockSpec((None, 2 * k, db2), lambda b, d: (b, 0, d)),
        out_shape=jax.ShapeDtypeStruct((B, 2 * k, D), jnp.float32),
        scratch_shapes=[
            pltpu.VMEM((2 * k, L), jnp.bfloat16),
            pltpu.VMEM((2 * k, L), jnp.bfloat16),
        ],
        interpret=interpret,
    )(idxt, x)
    if upto == 3:
        return sel

    y = pl.pallas_call(
        functools.partial(_inv_body, length=L, k=k),
        grid=(B, D // db2),
        in_specs=[
            pl.BlockSpec((B, k), lambda b, d: (0, 0)),
            pl.BlockSpec((None, 2 * k, db2), lambda b, d: (b, 0, d)),
        ],
        out_specs=pl.BlockSpec((None, L, db2), lambda b, d: (b, 0, d)),
        out_shape=jax.ShapeDtypeStruct((B, L, D), jnp.float32),
        scratch_shapes=[
            pltpu.VMEM((L, 2 * k), jnp.bfloat16),
            pltpu.VMEM((L, 2 * k), jnp.bfloat16),
        ],
        interpret=interpret,
    )(idx, sel)

    return y


def kernel(x):
    return _pipeline(x, 64)


# speedup vs baseline: 4.9874x; 1.1086x over previous
import functools

import numpy as np
import jax
import jax.numpy as jnp
from jax.experimental import pallas as pl
from jax.experimental.pallas import tpu as pltpu


def _dft_tables(lf_pad, lf, length):
    f = np.arange(lf_pad, dtype=np.int64)[:, None]
    t = np.arange(lf_pad, dtype=np.int64)[None, :]
    ang = (f * t % length).astype(np.float64) * (2.0 * np.pi / length)
    cre = np.cos(ang)
    cim = -np.sin(ang)
    cre[lf:] = 0.0
    cim[lf:] = 0.0
    cre[:, lf:] = 0.0
    cim[:, lf:] = 0.0
    cre[:, 0] *= 0.5
    if length // 2 < lf_pad:
        cre[:, length // 2] *= 0.5
    cim[:, 0] = 0.0
    if length // 2 < lf_pad:
        cim[:, length // 2] = 0.0

    def split(a):
        a32 = a.astype(np.float32)
        hi = a32.astype(jnp.bfloat16)
        lo = (a32 - hi.astype(np.float32)).astype(jnp.bfloat16)
        return jnp.asarray(hi), jnp.asarray(lo)

    return split(cre) + split(cim)


def _mag_body(cre_h_ref, cre_l_ref, cim_h_ref, cim_l_ref, xa_ref, xr_ref,
              out_ref, *, fb):
    i_f = pl.program_id(0)
    b = pl.program_id(1)
    i_d = pl.program_id(2)
    xa = xa_ref[...]
    xr = xr_ref[...]
    xe = xa + xr
    xo = xa - xr

    def mm3(h_ref, l_ref, v):
        vh = v.astype(jnp.bfloat16)
        vl = (v - vh.astype(jnp.float32)).astype(jnp.bfloat16)
        h = h_ref[...]
        out = jnp.dot(h, vh, preferred_element_type=jnp.float32)
        out += jnp.dot(h, vl, preferred_element_type=jnp.float32)
        out += jnp.dot(l_ref[...], vh, preferred_element_type=jnp.float32)
        return out

    xre = mm3(cre_h_ref, cre_l_ref, xe)
    xim = mm3(cim_h_ref, cim_l_ref, xo)
    part = jnp.sum(jnp.sqrt(xre * xre + xim * xim), axis=1)[None, None, :]

    @pl.when(i_d == 0)
    def _():
        out_ref[pl.ds(b, 1), pl.ds(i_f, 1), :] = part

    @pl.when(i_d != 0)
    def _():
        out_ref[pl.ds(b, 1), pl.ds(i_f, 1), :] += part


def _topk_body(mag_ref, idx_ref, idxt_ref, *, k):
    m = mag_ref[...]
    B = m.shape[0]
    cols = jax.lax.broadcasted_iota(jnp.int32, m.shape, 1)
    colsk = jax.lax.broadcasted_iota(jnp.int32, (B, k), 1)
    rowsk = jax.lax.broadcasted_iota(jnp.int32, (k, B), 0)
    sentinel = jnp.int32(m.shape[1])

    def body(j, carry):
        m, acc, acct = carry
        mx = jnp.max(m, axis=1, keepdims=True)
        idx = jnp.min(jnp.where(m == mx, cols, sentinel), axis=1)
        acc = jnp.where(colsk == j, idx[:, None], acc)
        acct = jnp.where(rowsk == j, idx[None, :], acct)
        m = jnp.where(cols == idx[:, None], -jnp.inf, m)
        return m, acc, acct

    _, acc, acct = jax.lax.fori_loop(
        0, k, body,
        (m, jnp.zeros((B, k), jnp.int32), jnp.zeros((k, B), jnp.int32)))
    idx_ref[...] = acc
    idxt_ref[...] = acct


def _split_bf16(v):
    vh = v.astype(jnp.bfloat16)
    vl = (v - vh.astype(jnp.float32)).astype(jnp.bfloat16)
    return vh, vl


def _dot3(c, x):
    ch, cl = _split_bf16(c)
    xh, xl = _split_bf16(x)
    out = jnp.dot(ch, xh, preferred_element_type=jnp.float32)
    out += jnp.dot(ch, xl, preferred_element_type=jnp.float32)
    out += jnp.dot(cl, xh, preferred_element_type=jnp.float32)
    return out


def _xsel_body(idxt_ref, x_ref, sel_ref, bh_ref, bl_ref, *, length, k):
    b = pl.program_id(0)

    @pl.when(pl.program_id(1) == 0)
    def _():
        idxt = idxt_ref[...]
        bcols = jax.lax.broadcasted_iota(jnp.int32, idxt.shape, 1)
        fj = jnp.sum(jnp.where(bcols == b, idxt, 0), axis=1, keepdims=True)
        fj2 = jnp.concatenate([fj, fj], axis=0)
        t = jax.lax.broadcasted_iota(jnp.int32, (1, length), 1)
        ang = ((fj2 * t) & (length - 1)).astype(jnp.float32) * (
            2.0 * np.pi / length)
        is_im = jax.lax.broadcasted_iota(jnp.int32, (2 * k, 1), 0) >= k
        c = jnp.where(is_im, -jnp.sin(ang), jnp.cos(ang))
        ch, cl = _split_bf16(c)
        bh_ref[...] = ch
        bl_ref[...] = cl

    xh, xl = _split_bf16(x_ref[...])
    ch = bh_ref[...]
    out = jnp.dot(ch, xh, preferred_element_type=jnp.float32)
    out += jnp.dot(ch, xl, preferred_element_type=jnp.float32)
    out += jnp.dot(bl_ref[...], xh, preferred_element_type=jnp.float32)
    sel_ref[...] = out


def _inv_body(idx_ref, sel_ref, y_ref, bh_ref, bl_ref, *, length, k):
    b = pl.program_id(0)

    @pl.when(pl.program_id(1) == 0)
    def _():
        idx = idx_ref[...]
        brows = jax.lax.broadcasted_iota(jnp.int32, idx.shape, 0)
        fj = jnp.sum(jnp.where(brows == b, idx, 0), axis=0, keepdims=True)
        fj2 = jnp.concatenate([fj, fj], axis=1)
        t = jax.lax.broadcasted_iota(jnp.int32, (length, 1), 0)
        ang = ((t * fj2) & (length - 1)).astype(jnp.float32) * (
            2.0 * np.pi / length)
        w = jnp.where((fj2 == 0) | (fj2 == length // 2), 0.5, 1.0) * (
            2.0 / length)
        is_im = jax.lax.broadcasted_iota(jnp.int32, (1, 2 * k), 1) >= k
        c = jnp.where(is_im, -jnp.sin(ang), jnp.cos(ang)) * w
        ch, cl = _split_bf16(c)
        bh_ref[...] = ch
        bl_ref[...] = cl

    xh, xl = _split_bf16(sel_ref[...])
    ch = bh_ref[...]
    out = jnp.dot(ch, xh, preferred_element_type=jnp.float32)
    out += jnp.dot(ch, xl, preferred_element_type=jnp.float32)
    out += jnp.dot(bl_ref[...], xh, preferred_element_type=jnp.float32)
    y_ref[...] = out


def _pipeline(x, k, interpret=False, upto=4):
    B, L, D = x.shape
    lf = L // 2 + 1
    lf_pad = -(-lf // 128) * 128
    nf = 2 if (lf_pad % 16 == 0 and lf_pad >= 2048) else 1
    fb = lf_pad // nf
    db1 = 256 if D % 256 == 0 else D
    db2 = 512 if D % 512 == 0 else D

    cre_h, cre_l, cim_h, cim_l = _dft_tables(lf_pad, lf, L)

    xr = jnp.take(x, (-jnp.arange(lf_pad)) % L, axis=1)

    basis_spec = pl.BlockSpec((fb, lf_pad), lambda f, b, d: (f, 0))
    xfold_spec = pl.BlockSpec((None, lf_pad, db1), lambda f, b, d: (b, 0, d))
    mag = pl.pallas_call(
        functools.partial(_mag_body, fb=fb),
        grid=(nf, B, D // db1),
        in_specs=[
            basis_spec, basis_spec, basis_spec, basis_spec,
            xfold_spec, xfold_spec,
        ],
        out_specs=pl.BlockSpec((B, nf, fb), lambda f, b, d: (0, 0, 0)),
        out_shape=jax.ShapeDtypeStruct((B, nf, fb), jnp.float32),
        interpret=interpret,
    )(cre_h, cre_l, cim_h, cim_l, x, xr)
    mag = mag.reshape(B, lf_pad)
    if upto == 1:
        return mag

    idx, idxt = pl.pallas_call(
        functools.partial(_topk_body, k=k),
        in_specs=[pl.BlockSpec((B, lf_pad), lambda: (0, 0))],
        out_specs=[
            pl.BlockSpec((B, k), lambda: (0, 0)),
            pl.BlockSpec((k, B), lambda: (0, 0)),
        ],
        out_shape=[
            jax.ShapeDtypeStruct((B, k), jnp.int32),
            jax.ShapeDtypeStruct((k, B), jnp.int32),
        ],
        interpret=interpret,
    )(mag)
    if upto == 2:
        return idx

    sel = pl.pallas_call(
        functools.partial(_xsel_body, length=L, k=k),
        grid=(B, D // db2),
        in_specs=[
            pl.BlockSpec((k, B), lambda b, d: (0, 0)),
            pl.BlockSpec((None, L, db2), lambda b, d: (b, 0, d)),
        ],
        out_specs=pl.BlockSpec((None, 2 * k, db2), lambda b, d: (b, 0, d)),
        out_shape=jax.ShapeDtypeStruct((B, 2 * k, D), jnp.float32),
        scratch_shapes=[
            pltpu.VMEM((2 * k, L), jnp.bfloat16),
            pltpu.VMEM((2 * k, L), jnp.bfloat16),
        ],
        interpret=interpret,
    )(idxt, x)
    if upto == 3:
        return sel

    y = pl.pallas_call(
        functools.partial(_inv_body, length=L, k=k),
        grid=(B, D // db2),
        in_specs=[
            pl.BlockSpec((B, k), lambda b, d: (0, 0)),
            pl.BlockSpec((None, 2 * k, db2), lambda b, d: (b, 0, d)),
        ],
        out_specs=pl.BlockSpec((None, L, db2), lambda b, d: (b, 0, d)),
        out_shape=jax.ShapeDtypeStruct((B, L, D), jnp.float32),
        scratch_shapes=[
            pltpu.VMEM((L, 2 * k), jnp.bfloat16),
            pltpu.VMEM((L, 2 * k), jnp.bfloat16),
        ],
        interpret=interpret,
    )(idx, sel)

    return y


def kernel(x):
    return _pipeline(x, 64)


# speedup vs baseline: 5.0384x; 1.0102x over previous
import functools

import numpy as np
import jax
import jax.numpy as jnp
from jax.experimental import pallas as pl
from jax.experimental.pallas import tpu as pltpu


def _dft_tables(lf_pad, lf, length):
    f = np.arange(lf_pad, dtype=np.int64)[:, None]
    t = np.arange(lf_pad, dtype=np.int64)[None, :]
    ang = (f * t % length).astype(np.float64) * (2.0 * np.pi / length)
    cre = np.cos(ang)
    cim = -np.sin(ang)
    cre[lf:] = 0.0
    cim[lf:] = 0.0
    cre[:, lf:] = 0.0
    cim[:, lf:] = 0.0
    cre[:, 0] *= 0.5
    if length // 2 < lf_pad:
        cre[:, length // 2] *= 0.5
    cim[:, 0] = 0.0
    if length // 2 < lf_pad:
        cim[:, length // 2] = 0.0

    def split(a):
        a32 = a.astype(np.float32)
        hi = a32.astype(jnp.bfloat16)
        lo = (a32 - hi.astype(np.float32)).astype(jnp.bfloat16)
        return jnp.asarray(hi), jnp.asarray(lo)

    return split(cre) + split(cim)


def _mag_body(cre_h_ref, cre_l_ref, cim_h_ref, cim_l_ref, xa_ref, xr_ref,
              out_ref, *, fb):
    i_f = pl.program_id(0)
    b = pl.program_id(1)
    i_d = pl.program_id(2)
    xa = xa_ref[...]
    xr = xr_ref[...]
    xe = xa + xr
    xo = xa - xr

    def mm3(h_ref, l_ref, v):
        vh = v.astype(jnp.bfloat16)
        vl = (v - vh.astype(jnp.float32)).astype(jnp.bfloat16)
        h = h_ref[...]
        out = jnp.dot(h, vh, preferred_element_type=jnp.float32)
        out += jnp.dot(h, vl, preferred_element_type=jnp.float32)
        out += jnp.dot(l_ref[...], vh, preferred_element_type=jnp.float32)
        return out

    xre = mm3(cre_h_ref, cre_l_ref, xe)
    xim = mm3(cim_h_ref, cim_l_ref, xo)
    part = jnp.sum(jnp.sqrt(xre * xre + xim * xim), axis=1)[None, None, :]

    @pl.when(i_d == 0)
    def _():
        out_ref[pl.ds(b, 1), pl.ds(i_f, 1), :] = part

    @pl.when(i_d != 0)
    def _():
        out_ref[pl.ds(b, 1), pl.ds(i_f, 1), :] += part


def _topk_body(mag_ref, idx_ref, idxt_ref, *, k):
    m = mag_ref[...]
    B = m.shape[0]
    cols = jax.lax.broadcasted_iota(jnp.int32, m.shape, 1)
    colsk = jax.lax.broadcasted_iota(jnp.int32, (B, k), 1)
    rowsk = jax.lax.broadcasted_iota(jnp.int32, (k, B), 0)
    sentinel = jnp.int32(m.shape[1])

    def body(j, carry):
        m, acc, acct = carry
        mx = jnp.max(m, axis=1, keepdims=True)
        idx = jnp.min(jnp.where(m == mx, cols, sentinel), axis=1)
        acc = jnp.where(colsk == j, idx[:, None], acc)
        acct = jnp.where(rowsk == j, idx[None, :], acct)
        m = jnp.where(cols == idx[:, None], -jnp.inf, m)
        return m, acc, acct

    _, acc, acct = jax.lax.fori_loop(
        0, k, body,
        (m, jnp.zeros((B, k), jnp.int32), jnp.zeros((k, B), jnp.int32)))
    idx_ref[...] = acc
    idxt_ref[...] = acct


def _split_bf16(v):
    vh = v.astype(jnp.bfloat16)
    vl = (v - vh.astype(jnp.float32)).astype(jnp.bfloat16)
    return vh, vl


def _dot3(c, x):
    ch, cl = _split_bf16(c)
    xh, xl = _split_bf16(x)
    out = jnp.dot(ch, xh, preferred_element_type=jnp.float32)
    out += jnp.dot(ch, xl, preferred_element_type=jnp.float32)
    out += jnp.dot(cl, xh, preferred_element_type=jnp.float32)
    return out


def _xsel_body(idxt_ref, x_ref, sel_ref, bh_ref, bl_ref, *, length, k):
    b = pl.program_id(0)

    @pl.when(pl.program_id(1) == 0)
    def _():
        idxt = idxt_ref[...]
        bcols = jax.lax.broadcasted_iota(jnp.int32, idxt.shape, 1)
        fj = jnp.sum(jnp.where(bcols == b, idxt, 0), axis=1, keepdims=True)
        fj2 = jnp.concatenate([fj, fj], axis=0)
        t = jax.lax.broadcasted_iota(jnp.int32, (1, length), 1)
        ang = ((fj2 * t) & (length - 1)).astype(jnp.float32) * (
            2.0 * np.pi / length)
        is_im = jax.lax.broadcasted_iota(jnp.int32, (2 * k, 1), 0) >= k
        c = jnp.where(is_im, -jnp.sin(ang), jnp.cos(ang))
        ch, cl = _split_bf16(c)
        bh_ref[...] = ch
        bl_ref[...] = cl

    xh, xl = _split_bf16(x_ref[...])
    ch = bh_ref[...]
    out = jnp.dot(ch, xh, preferred_element_type=jnp.float32)
    out += jnp.dot(ch, xl, preferred_element_type=jnp.float32)
    out += jnp.dot(bl_ref[...], xh, preferred_element_type=jnp.float32)
    sel_ref[...] = out


def _inv_body(idx_ref, sel_ref, y_ref, bh_ref, bl_ref, *, length, k):
    b = pl.program_id(0)

    @pl.when(pl.program_id(1) == 0)
    def _():
        idx = idx_ref[...]
        brows = jax.lax.broadcasted_iota(jnp.int32, idx.shape, 0)
        fj = jnp.sum(jnp.where(brows == b, idx, 0), axis=0, keepdims=True)
        fj2 = jnp.concatenate([fj, fj], axis=1)
        t = jax.lax.broadcasted_iota(jnp.int32, (length, 1), 0)
        ang = ((t * fj2) & (length - 1)).astype(jnp.float32) * (
            2.0 * np.pi / length)
        w = jnp.where((fj2 == 0) | (fj2 == length // 2), 0.5, 1.0) * (
            2.0 / length)
        is_im = jax.lax.broadcasted_iota(jnp.int32, (1, 2 * k), 1) >= k
        c = jnp.where(is_im, -jnp.sin(ang), jnp.cos(ang)) * w
        ch, cl = _split_bf16(c)
        bh_ref[...] = ch
        bl_ref[...] = cl

    xh, xl = _split_bf16(sel_ref[...])
    ch = bh_ref[...]
    out = jnp.dot(ch, xh, preferred_element_type=jnp.float32)
    out += jnp.dot(ch, xl, preferred_element_type=jnp.float32)
    out += jnp.dot(bl_ref[...], xh, preferred_element_type=jnp.float32)
    y_ref[...] = out


def _pipeline(x, k, interpret=False, upto=4):
    B, L, D = x.shape
    lf = L // 2 + 1
    lf_pad = -(-lf // 128) * 128
    nf = 2 if (lf_pad % 16 == 0 and lf_pad >= 2048) else 1
    fb = lf_pad // nf
    db1 = 512 if D % 512 == 0 else D
    db2 = 512 if D % 512 == 0 else D

    cre_h, cre_l, cim_h, cim_l = _dft_tables(lf_pad, lf, L)

    xr = jnp.take(x, (-jnp.arange(lf_pad)) % L, axis=1)

    basis_spec = pl.BlockSpec((fb, lf_pad), lambda f, b, d: (f, 0))
    xfold_spec = pl.BlockSpec((None, lf_pad, db1), lambda f, b, d: (b, 0, d))
    mag = pl.pallas_call(
        functools.partial(_mag_body, fb=fb),
        grid=(nf, B, D // db1),
        in_specs=[
            basis_spec, basis_spec, basis_spec, basis_spec,
            xfold_spec, xfold_spec,
        ],
        out_specs=pl.BlockSpec((B, nf, fb), lambda f, b, d: (0, 0, 0)),
        out_shape=jax.ShapeDtypeStruct((B, nf, fb), jnp.float32),
        compiler_params=pltpu.CompilerParams(vmem_limit_bytes=62 * 1024 * 1024),
        interpret=interpret,
    )(cre_h, cre_l, cim_h, cim_l, x, xr)
    mag = mag.reshape(B, lf_pad)
    if upto == 1:
        return mag

    idx, idxt = pl.pallas_call(
        functools.partial(_topk_body, k=k),
        in_specs=[pl.BlockSpec((B, lf_pad), lambda: (0, 0))],
        out_specs=[
            pl.BlockSpec((B, k), lambda: (0, 0)),
            pl.BlockSpec((k, B), lambda: (0, 0)),
        ],
        out_shape=[
            jax.ShapeDtypeStruct((B, k), jnp.int32),
            jax.ShapeDtypeStruct((k, B), jnp.int32),
        ],
        interpret=interpret,
    )(mag)
    if upto == 2:
        return idx

    sel = pl.pallas_call(
        functools.partial(_xsel_body, length=L, k=k),
        grid=(B, D // db2),
        in_specs=[
            pl.BlockSpec((k, B), lambda b, d: (0, 0)),
            pl.BlockSpec((None, L, db2), lambda b, d: (b, 0, d)),
        ],
        out_specs=pl.BlockSpec((None, 2 * k, db2), lambda b, d: (b, 0, d)),
        out_shape=jax.ShapeDtypeStruct((B, 2 * k, D), jnp.float32),
        scratch_shapes=[
            pltpu.VMEM((2 * k, L), jnp.bfloat16),
            pltpu.VMEM((2 * k, L), jnp.bfloat16),
        ],
        interpret=interpret,
    )(idxt, x)
    if upto == 3:
        return sel

    y = pl.pallas_call(
        functools.partial(_inv_body, length=L, k=k),
        grid=(B, D // db2),
        in_specs=[
            pl.BlockSpec((B, k), lambda b, d: (0, 0)),
            pl.BlockSpec((None, 2 * k, db2), lambda b, d: (b, 0, d)),
        ],
        out_specs=pl.BlockSpec((None, L, db2), lambda b, d: (b, 0, d)),
        out_shape=jax.ShapeDtypeStruct((B, L, D), jnp.float32),
        scratch_shapes=[
            pltpu.VMEM((L, 2 * k), jnp.bfloat16),
            pltpu.VMEM((L, 2 * k), jnp.bfloat16),
        ],
        interpret=interpret,
    )(idx, sel)

    return y


def kernel(x):
    return _pipeline(x, 64)
